# baseline (speedup 1.0000x reference)
import functools

import jax
import jax.numpy as jnp
from jax.experimental import pallas as pl
from jax.experimental.pallas import tpu as pltpu

_VMEM_LIMIT = 64 * 1024 * 1024


def _gather_kernel(ids_ref, tbl_ref, o_ref, *, rows, sc):
    c = pl.program_id(0)
    base = c * rows
    for mi in range(rows):
        i = pl.multiple_of(ids_ref[base + mi], sc)
        o_ref[pl.ds(sc * mi, sc), :] = tbl_ref[pl.ds(i, sc), :]


def _inproj_kernel(x_ref, w_ref, b_ref, o_ref, *, gsz, ts):
    nb, nd = x_ref.shape[1], x_ref.shape[2]
    xv = x_ref[...].reshape(ts * nb, nd).astype(jnp.bfloat16)
    out = (jnp.dot(xv, w_ref[...], preferred_element_type=jnp.float32)
           + b_ref[...]).astype(o_ref.dtype)
    o_ref[0] = out[:, :gsz].reshape(ts, nb, gsz)
    o_ref[1] = out[:, gsz:].reshape(ts, nb, gsz)


def _gru_kernel(xp_ref, wh_ref, bh_ref, len_ref, o_ref, h_ref, *, hsz, ts,
                steps):
    d = pl.program_id(0)
    tc = pl.program_id(1)

    @pl.when(tc == 0)
    def _():
        h_ref[...] = jnp.zeros_like(h_ref)

    wh = wh_ref[0]
    bh = bh_ref[0]
    lens = len_ref[...]
    h = h_ref[...]
    outs = []
    for j in range(ts):
        rj = j + d * (ts - 1 - 2 * j)
        sg = tc * ts + j
        t_step = sg + d * (steps - 1 - 2 * sg)
        gx = xp_ref[0, rj].astype(jnp.float32)
        gh = jnp.dot(h.astype(jnp.bfloat16), wh,
                     preferred_element_type=jnp.float32) + bh
        r = jax.nn.sigmoid(gx[:, :hsz] + gh[:, :hsz])
        z = jax.nn.sigmoid(gx[:, hsz:2 * hsz] + gh[:, hsz:2 * hsz])
        n = jnp.tanh(gx[:, 2 * hsz:] + r * gh[:, 2 * hsz:])
        h_new = (1.0 - z) * n + z * h
        valid = t_step < lens
        outs.append(jnp.where(valid, h_new, 0.0).astype(o_ref.dtype))
        h = jnp.where(valid, h_new, h)
    h_ref[...] = h

    @pl.when(d == 0)
    def _():
        for j in range(ts):
            o_ref[0, :, j, :] = outs[j]

    @pl.when(d == 1)
    def _():
        for j in range(ts):
            o_ref[0, :, ts - 1 - j, :] = outs[j]


def _combine_kernel(y_ref, o_ref, *, eps):
    x = (y_ref[0].astype(jnp.float32) + y_ref[1].astype(jnp.float32)) * 0.5
    norm = jnp.sqrt(jnp.sum(x * x, axis=-1, keepdims=True)) + eps
    o_ref[...] = x * pl.reciprocal(norm, approx=False)


def kernel(embedding, ids, lengths,
           l0d0_w_ih, l0d0_b_ih, l0d0_w_hh, l0d0_b_hh,
           l0d1_w_ih, l0d1_b_ih, l0d1_w_hh, l0d1_b_hh):
    B, T = ids.shape
    V, D = embedding.shape
    G, H = l0d0_w_hh.shape
    M = T * B
    S = D // 128

    ids_tb = (jnp.transpose(ids).reshape(M) * S).astype(jnp.int32)
    tbl = embedding.reshape(V * S, 128)

    x = pl.pallas_call(
        functools.partial(_gather_kernel, rows=B, sc=S),
        out_shape=jax.ShapeDtypeStruct((M * S, 128), jnp.float32),
        grid_spec=pltpu.PrefetchScalarGridSpec(
            num_scalar_prefetch=1,
            grid=(T,),
            in_specs=[pl.BlockSpec((V * S, 128), lambda c, ids: (0, 0))],
            out_specs=pl.BlockSpec((B * S, 128), lambda c, ids: (c, 0))),
        compiler_params=pltpu.CompilerParams(
            dimension_semantics=("parallel",),
            vmem_limit_bytes=_VMEM_LIMIT),
    )(ids_tb, tbl)
    x = x.reshape(T, B, D)

    w_cat = jnp.concatenate(
        [l0d0_w_ih.T, l0d1_w_ih.T], axis=1).astype(jnp.bfloat16)
    b_cat = jnp.concatenate([l0d0_b_ih, l0d1_b_ih]).reshape(1, 2 * G)
    ts_i = 4
    xp = pl.pallas_call(
        functools.partial(_inproj_kernel, gsz=G, ts=ts_i),
        out_shape=jax.ShapeDtypeStruct((2, T, B, G), jnp.bfloat16),
        grid_spec=pltpu.PrefetchScalarGridSpec(
            num_scalar_prefetch=0,
            grid=(T // ts_i,),
            in_specs=[pl.BlockSpec((ts_i, B, D), lambda t: (t, 0, 0)),
                      pl.BlockSpec((D, 2 * G), lambda t: (0, 0)),
                      pl.BlockSpec((1, 2 * G), lambda t: (0, 0))],
            out_specs=pl.BlockSpec((2, ts_i, B, G), lambda t: (0, t, 0, 0))),
        compiler_params=pltpu.CompilerParams(
            dimension_semantics=("parallel",),
            vmem_limit_bytes=_VMEM_LIMIT),
    )(x, w_cat, b_cat)

    wh = jnp.stack([l0d0_w_hh.T, l0d1_w_hh.T]).astype(jnp.bfloat16)
    bh = jnp.stack([l0d0_b_hh, l0d1_b_hh]).reshape(2, 1, G)
    len2 = lengths.astype(jnp.int32).reshape(B, 1)
    ts_g = 8
    ntc = T // ts_g
    tc_eff = lambda d, tc: tc + d * (ntc - 1 - 2 * tc)
    y = pl.pallas_call(
        functools.partial(_gru_kernel, hsz=H, ts=ts_g, steps=T),
        out_shape=jax.ShapeDtypeStruct((2, B, T, H), jnp.bfloat16),
        grid_spec=pltpu.PrefetchScalarGridSpec(
            num_scalar_prefetch=0,
            grid=(2, ntc),
            in_specs=[
                pl.BlockSpec((1, ts_g, B, G),
                             lambda d, tc: (d, tc_eff(d, tc), 0, 0)),
                pl.BlockSpec((1, H, G), lambda d, tc: (d, 0, 0)),
                pl.BlockSpec((1, 1, G), lambda d, tc: (d, 0, 0)),
                pl.BlockSpec((B, 1), lambda d, tc: (0, 0))],
            out_specs=pl.BlockSpec((1, B, ts_g, H),
                                   lambda d, tc: (d, 0, tc_eff(d, tc), 0)),
            scratch_shapes=[pltpu.VMEM((B, H), jnp.float32)]),
        compiler_params=pltpu.CompilerParams(
            dimension_semantics=("parallel", "arbitrary"),
            vmem_limit_bytes=_VMEM_LIMIT),
    )(xp, wh, bh, len2)

    tb = min(128, B)
    cap = pl.pallas_call(
        functools.partial(_combine_kernel, eps=1e-8),
        out_shape=jax.ShapeDtypeStruct((B, T, H), jnp.float32),
        grid_spec=pltpu.PrefetchScalarGridSpec(
            num_scalar_prefetch=0,
            grid=(B // tb, ntc),
            in_specs=[pl.BlockSpec((2, tb, ts_g, H),
                                   lambda i, tc: (0, i, tc, 0))],
            out_specs=pl.BlockSpec((tb, ts_g, H), lambda i, tc: (i, tc, 0))),
        compiler_params=pltpu.CompilerParams(
            dimension_semantics=("parallel", "arbitrary"),
            vmem_limit_bytes=_VMEM_LIMIT),
    )(y)

    return cap, lengths
